# baseline (speedup 1.0000x reference)
import functools

import jax
import jax.numpy as jnp
import numpy as np
from jax import lax
from jax.experimental import pallas as pl
from jax.experimental.pallas import tpu as pltpu
from jax.experimental.pallas import tpu_sc as plsc

B = 16384
D = 64
BN_EPS = 1e-5

NC = 2
NS = 16
NW = NC * NS
BPW = B // NW
CH = 128
NCH = BPW // CH

NU = 1000000
NM = 100000
NBU = NU // 128
NBM = NM // 128


@functools.lru_cache(maxsize=None)
def _make_sc_transpose():
    mesh = plsc.VectorSubcoreMesh(core_axis_name="c", subcore_axis_name="s")

    @functools.partial(
        pl.kernel,
        mesh=mesh,
        compiler_params=pltpu.CompilerParams(needs_layout_passes=False,
                                             disable_bounds_checks=True),
        out_type=[
            jax.ShapeDtypeStruct((NU // 2, 128), jnp.float32),
            jax.ShapeDtypeStruct((NM // 2, 128), jnp.float32),
        ],
        scratch_types=[
            pltpu.VMEM((D, 128), jnp.float32),
            pltpu.VMEM((D, 128), jnp.float32),
            pltpu.VMEM((D, 128), jnp.float32),
            pltpu.VMEM((D, 128), jnp.float32),
            pltpu.VMEM((D, 128), jnp.float32),
            pltpu.SemaphoreType.DMA,
            pltpu.SemaphoreType.DMA,
            pltpu.SemaphoreType.DMA,
            pltpu.SemaphoreType.DMA,
        ],
    )
    def _sc_xpose(utT_hbm, mtT_hbm, utp_hbm, mtp_hbm, ibuf0, ibuf1,
                  obuf0, obuf1, tbuf, isem0, isem1, osem0, osem1):
        wid = lax.axis_index("s") * NC + lax.axis_index("c")
        lanes = lax.iota(jnp.int32, 16)
        ibufs = (ibuf0, ibuf1)
        obufs = (obuf0, obuf1)
        isems = (isem0, isem1)
        osems = (osem0, osem1)

        def one_table(tbl_hbm, out_hbm, nblk):
            def fire_in(t, buf):
                col0 = pl.multiple_of(col0_of(t), 128)
                pltpu.async_copy(tbl_hbm.at[:, pl.ds(col0, 128)],
                                 ibufs[buf], isems[buf])

            def col0_of(t):
                return (wid + t * NW) * 128

            def _out_row(t):
                return pl.multiple_of((wid + t * NW) * 64, 64)

            def out_desc(t, buf):
                return pltpu.make_async_copy(
                    obufs[buf], out_hbm.at[pl.ds(_out_row(t), 64)],
                    osems[buf])

            def xpose_block(t, buf):
                pltpu.make_async_copy(tbl_hbm.at[:, pl.ds(0, 128)],
                                      ibufs[buf], isems[buf]).wait()

                @plsc.parallel_loop(0, D, unroll=8)
                def _(j):
                    for k in range(8):
                        rowv = lanes + 16 * (k % 4)
                        colv = jnp.full((16,), 0, jnp.int32) + (
                            2 * j + (1 if k >= 4 else 0))
                        vals = plsc.load_gather(ibufs[buf], [rowv, colv])
                        obufs[buf][j, pl.ds(k * 16, 16)] = vals

            def step(t, buf):
                @pl.when(wid + (t + 1) * NW < nblk)
                def _():
                    fire_in(t + 1, 1 - buf)

                @pl.when(t >= 2)
                def _():
                    out_desc(t, buf).wait()

                xpose_block(t, buf)
                pltpu.async_copy(obufs[buf],
                                 out_hbm.at[pl.ds(_out_row(t), 64)],
                                 osems[buf])

            @pl.when(wid < nblk)
            def _():
                fire_in(0, 0)

            def body(p, _):
                t0 = 2 * p

                @pl.when(wid + t0 * NW < nblk)
                def _():
                    step(t0, 0)

                t1 = 2 * p + 1

                @pl.when(wid + t1 * NW < nblk)
                def _():
                    step(t1, 1)

                return 0

            niter = (nblk + NW - 1) // NW
            lax.fori_loop(0, (niter + 1) // 2, body, 0)
            nt = lax.max((nblk - wid + NW - 1) // NW, 0)

            @pl.when((nt + 1) // 2 >= 1)
            def _():
                out_desc(0, 0).wait()

            @pl.when(nt // 2 >= 1)
            def _():
                out_desc(0, 1).wait()

        one_table(utT_hbm, utp_hbm, NBU)
        one_table(mtT_hbm, mtp_hbm, NBM)

        @pl.when(wid == 0)
        def _():
            st = pl.multiple_of(jnp.int32(NBU) * 128, 128)
            pltpu.sync_copy(utT_hbm.at[:, pl.ds(st, 128)], tbuf)

            def jbody(j, _):
                for k in range(8):
                    rowv = lanes + 16 * (k % 4)
                    colv = jnp.full((16,), 0, jnp.int32) + (
                        2 * j + (1 if k >= 4 else 0))
                    vals = plsc.load_gather(tbuf, [rowv, colv])
                    obufs[0][j, pl.ds(k * 16, 16)] = vals
                return 0

            lax.fori_loop(0, 32, jbody, 0)
            pltpu.sync_copy(obufs[0].at[pl.ds(0, 32), :],
                            utp_hbm.at[pl.ds(NU // 2 - 32, 32)])

        @pl.when(wid == 1)
        def _():
            st = pl.multiple_of(jnp.int32(NBM) * 128, 128)
            pltpu.sync_copy(mtT_hbm.at[:, pl.ds(st, 128)], tbuf)

            def jbody(j, _):
                for k in range(8):
                    rowv = lanes + 16 * (k % 4)
                    colv = jnp.full((16,), 0, jnp.int32) + (
                        2 * j + (1 if k >= 4 else 0))
                    vals = plsc.load_gather(tbuf, [rowv, colv])
                    obufs[0][j, pl.ds(k * 16, 16)] = vals
                return 0

            lax.fori_loop(0, 16, jbody, 0)
            pltpu.sync_copy(obufs[0].at[pl.ds(0, 16), :],
                            mtp_hbm.at[pl.ds(NM // 2 - 16, 16)])

    return _sc_xpose


@functools.lru_cache(maxsize=None)
def _make_sc_gather():
    mesh = plsc.VectorSubcoreMesh(core_axis_name="c", subcore_axis_name="s")

    @functools.partial(
        pl.kernel,
        mesh=mesh,
        compiler_params=pltpu.CompilerParams(needs_layout_passes=False),
        out_type=[
            jax.ShapeDtypeStruct((NW, D, BPW), jnp.float32),
            jax.ShapeDtypeStruct((NW, D, BPW), jnp.float32),
        ],
        scratch_types=[
            pltpu.VMEM((BPW,), jnp.int32),
            pltpu.VMEM((BPW,), jnp.int32),
            pltpu.VMEM((CH,), jnp.int32),
            pltpu.VMEM((CH,), jnp.int32),
            pltpu.VMEM((CH, 2 * D), jnp.float32),
            pltpu.VMEM((CH, 2 * D), jnp.float32),
            pltpu.VMEM((D, BPW), jnp.float32),
            pltpu.SemaphoreType.DMA,
            pltpu.SemaphoreType.DMA,
        ],
    )
    def _sc_gather(users_hbm, movies_hbm, ut_hbm, mt_hbm, ueT_hbm, meT_hbm,
                   idx_u, idx_m, ig0, ig1, rows0, rows1, out_T, sem0, sem1):
        wid = lax.axis_index("s") * NC + lax.axis_index("c")
        pltpu.sync_copy(users_hbm.at[wid], idx_u)
        pltpu.sync_copy(movies_hbm.at[wid], idx_m)
        igs = (ig0, ig1)
        rows = (rows0, rows1)
        sems = (sem0, sem1)
        lanes = lax.iota(jnp.int32, 16)

        def one_table(idx_ref, tbl_hbm, out_hbm):
            def fire(ch, buf):
                def fv(v, _):
                    igs[buf][pl.ds(v * 16, 16)] = lax.shift_right_logical(
                        idx_ref[pl.ds(ch * CH + v * 16, 16)], 1)
                    return 0

                lax.fori_loop(0, CH // 16, fv, 0)
                pltpu.async_copy(tbl_hbm.at[igs[buf]], rows[buf], sems[buf])

            def drain_select(ch, buf):
                pltpu.make_async_copy(tbl_hbm.at[igs[buf]], rows[buf],
                                      sems[buf]).wait()
                base = ch * CH

                def v_body(v, _):
                    iv = idx_ref[pl.ds(base + v * 16, 16)]
                    hv = lax.bitwise_and(iv, 1) * D
                    rowv = lanes + v * 16

                    def col_body(c, _):
                        colv = hv + jnp.full((16,), 0, jnp.int32) + c
                        vals = plsc.load_gather(rows[buf], [rowv, colv])
                        out_T[c, pl.ds(base + v * 16, 16)] = vals
                        return 0

                    lax.fori_loop(0, D, col_body, 0)
                    return 0

                lax.fori_loop(0, CH // 16, v_body, 0)

            fire(0, 0)
            for ch in range(NCH):
                if ch + 1 < NCH:
                    fire(ch + 1, (ch + 1) % 2)
                drain_select(ch, ch % 2)
            pltpu.sync_copy(out_T, out_hbm.at[wid])

        one_table(idx_u, ut_hbm, ueT_hbm)
        one_table(idx_m, mt_hbm, meT_hbm)

    return _sc_gather


def _mlp_body(ueT_ref, meT_ref, w1_ref, c1_ref, w2_ref, c2_ref, w3_ref,
              c3_ref, out_ref):
    w1 = w1_ref[...]
    tn = (((1,), (0,)), ((), ()))
    h = lax.dot_general(w1[:, :D], ueT_ref[0], tn,
                        preferred_element_type=jnp.float32)
    h += lax.dot_general(w1[:, D:], meT_ref[0], tn,
                         preferred_element_type=jnp.float32)
    h = jnp.maximum(h + c1_ref[...], 0.0)
    h = lax.dot_general(w2_ref[...], h, tn, preferred_element_type=jnp.float32)
    h = jnp.maximum(h + c2_ref[...], 0.0)
    o = jnp.sum(h * w3_ref[...], axis=0, keepdims=True) + c3_ref[...]
    out_ref[...] = 1.0 / (1.0 + jnp.exp(-o))


def kernel(users, movies, user_table, movie_table,
           W1, b1, g1, be1, W2, b2, g2, be2, W3, b3):
    u = users.astype(jnp.int32).reshape(NW, BPW)
    m = movies.astype(jnp.int32).reshape(NW, BPW)
    utp, mtp = _make_sc_transpose()(user_table.T, movie_table.T)
    ueT, meT = _make_sc_gather()(u, m, utp, mtp)

    s = np.float32(1.0 / np.sqrt(1.0 + BN_EPS))
    w1 = W1 * (g1 * s)[:, None]
    c1 = (b1 * g1 * s + be1).reshape(32, 1)
    w2 = W2 * (g2 * s)[:, None]
    c2 = (b2 * g2 * s + be2).reshape(16, 1)
    w3 = W3.reshape(16, 1)
    c3 = b3.reshape(1, 1)

    outT = pl.pallas_call(
        _mlp_body,
        grid=(NW,),
        in_specs=[
            pl.BlockSpec((1, D, BPW), lambda w: (w, 0, 0)),
            pl.BlockSpec((1, D, BPW), lambda w: (w, 0, 0)),
            pl.BlockSpec((32, 128), lambda w: (0, 0)),
            pl.BlockSpec((32, 1), lambda w: (0, 0)),
            pl.BlockSpec((16, 32), lambda w: (0, 0)),
            pl.BlockSpec((16, 1), lambda w: (0, 0)),
            pl.BlockSpec((16, 1), lambda w: (0, 0)),
            pl.BlockSpec((1, 1), lambda w: (0, 0)),
        ],
        out_specs=pl.BlockSpec((1, BPW), lambda w: (0, w)),
        out_shape=jax.ShapeDtypeStruct((1, B), jnp.float32),
    )(ueT, meT, w1, c1, w2, c2, w3, c3)
    return outT.reshape(B, 1)


# speedup vs baseline: 2.0000x; 2.0000x over previous
import functools

import jax
import jax.numpy as jnp
import numpy as np
from jax import lax
from jax.experimental import pallas as pl
from jax.experimental.pallas import tpu as pltpu
from jax.experimental.pallas import tpu_sc as plsc

B = 16384
D = 64
BN_EPS = 1e-5

NC = 2
NS = 16
NW = NC * NS
BPW = B // NW
CHUNK = 32


@functools.lru_cache(maxsize=None)
def _make_sc_gather(nu, nm):
    mesh = plsc.VectorSubcoreMesh(core_axis_name="c", subcore_axis_name="s")

    @functools.partial(
        pl.kernel,
        mesh=mesh,
        compiler_params=pltpu.CompilerParams(needs_layout_passes=False),
        out_type=[
            jax.ShapeDtypeStruct((NW, BPW, D), jnp.float32),
            jax.ShapeDtypeStruct((NW, BPW, D), jnp.float32),
        ],
        scratch_types=[
            pltpu.VMEM((BPW,), jnp.int32),
            pltpu.VMEM((BPW,), jnp.int32),
            pltpu.VMEM((CHUNK, 8, D), jnp.float32),
            pltpu.VMEM((BPW, D), jnp.float32),
            pltpu.SemaphoreType.DMA,
            pltpu.SemaphoreType.DMA,
        ],
    )
    def _sc_gather(users_hbm, movies_hbm, ut_hbm, mt_hbm, ue_hbm, me_hbm,
                   idx_u, idx_m, rows_g, out_rows, sem0, sem1):
        wid = lax.axis_index("s") * NC + lax.axis_index("c")
        pltpu.sync_copy(users_hbm.at[wid], idx_u)
        pltpu.sync_copy(movies_hbm.at[wid], idx_m)
        sems = (sem0, sem1)
        NG = BPW // 16

        def one_table(idx_ref, tbl_hbm, out_hbm):
            def fire(gi, half):
                iv = idx_ref[pl.ds(gi * 16, 16)]
                gv = lax.shift_right_logical(iv, 3)
                for j in range(16):
                    start = pl.multiple_of(gv[j] * 8, 8)
                    pltpu.async_copy(tbl_hbm.at[pl.ds(start, 8)],
                                     rows_g.at[half * 16 + j], sems[half])

            def drain_process(gi, half):
                base = gi * 16
                iv = idx_ref[pl.ds(base, 16)]
                sv = lax.bitwise_and(iv, 7)
                for j in range(16):
                    slot = half * 16 + j
                    pltpu.make_async_copy(tbl_hbm.at[pl.ds(0, 8)],
                                          rows_g.at[slot], sems[half]).wait()
                    s = sv[j]
                    for c in range(D // 16):
                        out_rows[base + j, pl.ds(c * 16, 16)] = (
                            rows_g[slot, s, pl.ds(c * 16, 16)])

            fire(0, 0)

            def body(p, _):
                gi_a = 2 * p + 1

                @pl.when(gi_a < NG)
                def _():
                    fire(gi_a, 1)

                drain_process(2 * p, 0)
                gi_b = 2 * p + 2

                @pl.when(gi_b < NG)
                def _():
                    fire(gi_b, 0)

                drain_process(2 * p + 1, 1)
                return 0

            lax.fori_loop(0, NG // 2, body, 0)
            pltpu.sync_copy(out_rows, out_hbm.at[wid])

        one_table(idx_u, ut_hbm, ue_hbm)
        one_table(idx_m, mt_hbm, me_hbm)

    return _sc_gather


def _mlp_body(ue_ref, me_ref, w1_ref, c1_ref, w2_ref, c2_ref, w3_ref, c3_ref,
              out_ref):
    w1 = w1_ref[...]
    nt = (((1,), (1,)), ((), ()))
    h = lax.dot_general(ue_ref[0], w1[:, :D], nt,
                        preferred_element_type=jnp.float32)
    h += lax.dot_general(me_ref[0], w1[:, D:], nt,
                         preferred_element_type=jnp.float32)
    h = jnp.maximum(h + c1_ref[...], 0.0)
    h = lax.dot_general(h, w2_ref[...], nt, preferred_element_type=jnp.float32)
    h = jnp.maximum(h + c2_ref[...], 0.0)
    o = jnp.sum(h * w3_ref[...], axis=1, keepdims=True) + c3_ref[...]
    out_ref[0] = 1.0 / (1.0 + jnp.exp(-o))


def kernel(users, movies, user_table, movie_table,
           W1, b1, g1, be1, W2, b2, g2, be2, W3, b3):
    u = users.astype(jnp.int32).reshape(NW, BPW)
    m = movies.astype(jnp.int32).reshape(NW, BPW)
    ue3, me3 = _make_sc_gather(user_table.shape[0], movie_table.shape[0])(
        u, m, user_table, movie_table)

    s = np.float32(1.0 / np.sqrt(1.0 + BN_EPS))
    w1 = W1 * (g1 * s)[:, None]
    c1 = (b1 * g1 * s + be1).reshape(1, 32)
    w2 = W2 * (g2 * s)[:, None]
    c2 = (b2 * g2 * s + be2).reshape(1, 16)
    w3 = W3.reshape(1, 16)
    c3 = b3.reshape(1, 1)

    out = pl.pallas_call(
        _mlp_body,
        grid=(NW,),
        in_specs=[
            pl.BlockSpec((1, BPW, D), lambda w: (w, 0, 0)),
            pl.BlockSpec((1, BPW, D), lambda w: (w, 0, 0)),
            pl.BlockSpec((32, 128), lambda w: (0, 0)),
            pl.BlockSpec((1, 32), lambda w: (0, 0)),
            pl.BlockSpec((16, 32), lambda w: (0, 0)),
            pl.BlockSpec((1, 16), lambda w: (0, 0)),
            pl.BlockSpec((1, 16), lambda w: (0, 0)),
            pl.BlockSpec((1, 1), lambda w: (0, 0)),
        ],
        out_specs=pl.BlockSpec((1, BPW, 1), lambda w: (w, 0, 0)),
        out_shape=jax.ShapeDtypeStruct((NW, BPW, 1), jnp.float32),
    )(ue3, me3, w1, c1, w2, c2, w3, c3)
    return out.reshape(B, 1)


# speedup vs baseline: 2.7363x; 1.3682x over previous
import functools

import jax
import jax.numpy as jnp
import numpy as np
from jax import lax
from jax.experimental import pallas as pl
from jax.experimental.pallas import tpu as pltpu
from jax.experimental.pallas import tpu_sc as plsc

B = 16384
D = 64
BN_EPS = 1e-5

NC = 2
NS = 16
NW = NC * NS
BPW = B // NW
CHUNK = 32


@functools.lru_cache(maxsize=None)
def _make_sc_gather(nu, nm):
    mesh = plsc.VectorSubcoreMesh(core_axis_name="c", subcore_axis_name="s")

    @functools.partial(
        pl.kernel,
        mesh=mesh,
        compiler_params=pltpu.CompilerParams(needs_layout_passes=False),
        out_type=[
            jax.ShapeDtypeStruct((NW, BPW, D), jnp.float32),
            jax.ShapeDtypeStruct((NW, BPW, D), jnp.float32),
        ],
        scratch_types=[
            pltpu.VMEM((BPW,), jnp.int32),
            pltpu.VMEM((BPW,), jnp.int32),
            pltpu.VMEM((CHUNK, 8, D), jnp.float32),
            pltpu.VMEM((BPW, D), jnp.float32),
            pltpu.SemaphoreType.DMA,
            pltpu.SemaphoreType.DMA,
        ],
    )
    def _sc_gather(users_hbm, movies_hbm, ut_hbm, mt_hbm, ue_hbm, me_hbm,
                   idx_u, idx_m, rows_g, out_rows, sem0, sem1):
        wid = lax.axis_index("s") * NC + lax.axis_index("c")
        pltpu.sync_copy(users_hbm.at[wid], idx_u)
        pltpu.sync_copy(movies_hbm.at[wid], idx_m)
        sems = (sem0, sem1)
        NG = BPW // 16

        def one_table(idx_ref, tbl_hbm, out_hbm):
            def fire(gi, half):
                iv = idx_ref[pl.ds(gi * 16, 16)]
                gv = lax.shift_right_logical(iv, 3)
                for j in range(16):
                    pltpu.async_copy(tbl_hbm.at[gv[j]],
                                     rows_g.at[half * 16 + j], sems[half])

            def drain_process(gi, half):
                base = gi * 16
                iv = idx_ref[pl.ds(base, 16)]
                sv = lax.bitwise_and(iv, 7)
                for j in range(16):
                    slot = half * 16 + j
                    pltpu.make_async_copy(tbl_hbm.at[0],
                                          rows_g.at[slot], sems[half]).wait()
                    s = sv[j]
                    for c in range(D // 16):
                        out_rows[base + j, pl.ds(c * 16, 16)] = (
                            rows_g[slot, s, pl.ds(c * 16, 16)])

            fire(0, 0)

            def body(p, _):
                gi_a = 2 * p + 1

                @pl.when(gi_a < NG)
                def _():
                    fire(gi_a, 1)

                drain_process(2 * p, 0)
                gi_b = 2 * p + 2

                @pl.when(gi_b < NG)
                def _():
                    fire(gi_b, 0)

                drain_process(2 * p + 1, 1)
                return 0

            lax.fori_loop(0, NG // 2, body, 0)
            pltpu.sync_copy(out_rows, out_hbm.at[wid])

        one_table(idx_u, ut_hbm, ue_hbm)
        one_table(idx_m, mt_hbm, me_hbm)

    return _sc_gather


def _mlp_body(ue_ref, me_ref, w1_ref, c1_ref, w2_ref, c2_ref, w3_ref, c3_ref,
              out_ref):
    w1 = w1_ref[...]
    nt = (((1,), (1,)), ((), ()))
    h = lax.dot_general(ue_ref[0], w1[:, :D], nt,
                        preferred_element_type=jnp.float32)
    h += lax.dot_general(me_ref[0], w1[:, D:], nt,
                         preferred_element_type=jnp.float32)
    h = jnp.maximum(h + c1_ref[...], 0.0)
    h = lax.dot_general(h, w2_ref[...], nt, preferred_element_type=jnp.float32)
    h = jnp.maximum(h + c2_ref[...], 0.0)
    o = jnp.sum(h * w3_ref[...], axis=1, keepdims=True) + c3_ref[...]
    out_ref[0] = 1.0 / (1.0 + jnp.exp(-o))


def kernel(users, movies, user_table, movie_table,
           W1, b1, g1, be1, W2, b2, g2, be2, W3, b3):
    u = users.astype(jnp.int32).reshape(NW, BPW)
    m = movies.astype(jnp.int32).reshape(NW, BPW)
    ut3 = user_table.reshape(user_table.shape[0] // 8, 8, D)
    mt3 = movie_table.reshape(movie_table.shape[0] // 8, 8, D)
    ue3, me3 = _make_sc_gather(ut3.shape[0], mt3.shape[0])(u, m, ut3, mt3)

    s = np.float32(1.0 / np.sqrt(1.0 + BN_EPS))
    w1 = W1 * (g1 * s)[:, None]
    c1 = (b1 * g1 * s + be1).reshape(1, 32)
    w2 = W2 * (g2 * s)[:, None]
    c2 = (b2 * g2 * s + be2).reshape(1, 16)
    w3 = W3.reshape(1, 16)
    c3 = b3.reshape(1, 1)

    out = pl.pallas_call(
        _mlp_body,
        grid=(NW,),
        in_specs=[
            pl.BlockSpec((1, BPW, D), lambda w: (w, 0, 0)),
            pl.BlockSpec((1, BPW, D), lambda w: (w, 0, 0)),
            pl.BlockSpec((32, 128), lambda w: (0, 0)),
            pl.BlockSpec((1, 32), lambda w: (0, 0)),
            pl.BlockSpec((16, 32), lambda w: (0, 0)),
            pl.BlockSpec((1, 16), lambda w: (0, 0)),
            pl.BlockSpec((1, 16), lambda w: (0, 0)),
            pl.BlockSpec((1, 1), lambda w: (0, 0)),
        ],
        out_specs=pl.BlockSpec((1, BPW, 1), lambda w: (w, 0, 0)),
        out_shape=jax.ShapeDtypeStruct((NW, BPW, 1), jnp.float32),
    )(ue3, me3, w1, c1, w2, c2, w3, c3)
    return out.reshape(B, 1)


# speedup vs baseline: 2.7401x; 1.0014x over previous
import functools

import jax
import jax.numpy as jnp
import numpy as np
from jax import lax
from jax.experimental import pallas as pl
from jax.experimental.pallas import tpu as pltpu
from jax.experimental.pallas import tpu_sc as plsc

B = 16384
D = 64
BN_EPS = 1e-5

NC = 2
NS = 16
NW = NC * NS
BPW = B // NW
CHUNK = 32


@functools.lru_cache(maxsize=None)
def _make_sc_gather(nu, nm):
    mesh = plsc.VectorSubcoreMesh(core_axis_name="c", subcore_axis_name="s")

    @functools.partial(
        pl.kernel,
        mesh=mesh,
        compiler_params=pltpu.CompilerParams(needs_layout_passes=False),
        out_type=[
            jax.ShapeDtypeStruct((NW, BPW, D), jnp.float32),
            jax.ShapeDtypeStruct((NW, BPW, D), jnp.float32),
        ],
        scratch_types=[
            pltpu.VMEM((BPW,), jnp.int32),
            pltpu.VMEM((BPW,), jnp.int32),
            pltpu.VMEM((CHUNK, 8, D), jnp.float32),
            pltpu.VMEM((BPW, D), jnp.float32),
            pltpu.SemaphoreType.DMA,
            pltpu.SemaphoreType.DMA,
        ],
    )
    def _sc_gather(users_hbm, movies_hbm, ut_hbm, mt_hbm, ue_hbm, me_hbm,
                   idx_u, idx_m, rows_g, out_rows, sem0, sem1):
        wid = lax.axis_index("s") * NC + lax.axis_index("c")
        base = pl.multiple_of(wid * BPW, BPW)
        pltpu.sync_copy(users_hbm.at[pl.ds(base, BPW)], idx_u)
        pltpu.sync_copy(movies_hbm.at[pl.ds(base, BPW)], idx_m)
        sems = (sem0, sem1)
        NG = BPW // 16

        def one_table(idx_ref, tbl_hbm, out_hbm):
            def fire(gi, half):
                iv = idx_ref[pl.ds(gi * 16, 16)]
                gv = lax.shift_right_logical(iv, 3)
                for j in range(16):
                    pltpu.async_copy(tbl_hbm.at[gv[j]],
                                     rows_g.at[half * 16 + j], sems[half])

            def drain_process(gi, half):
                base = gi * 16
                iv = idx_ref[pl.ds(base, 16)]
                sv = lax.bitwise_and(iv, 7)
                for j in range(16):
                    slot = half * 16 + j
                    pltpu.make_async_copy(tbl_hbm.at[0],
                                          rows_g.at[slot], sems[half]).wait()
                    s = sv[j]
                    for c in range(D // 16):
                        out_rows[base + j, pl.ds(c * 16, 16)] = (
                            rows_g[slot, s, pl.ds(c * 16, 16)])

            fire(0, 0)

            def body(p, _):
                gi_a = 2 * p + 1

                @pl.when(gi_a < NG)
                def _():
                    fire(gi_a, 1)

                drain_process(2 * p, 0)
                gi_b = 2 * p + 2

                @pl.when(gi_b < NG)
                def _():
                    fire(gi_b, 0)

                drain_process(2 * p + 1, 1)
                return 0

            lax.fori_loop(0, NG // 2, body, 0)
            pltpu.sync_copy(out_rows, out_hbm.at[wid])

        one_table(idx_u, ut_hbm, ue_hbm)
        one_table(idx_m, mt_hbm, me_hbm)

    return _sc_gather


def _mlp_body(ue_ref, me_ref, w1_ref, c1_ref, w2_ref, c2_ref, w3_ref, c3_ref,
              out_ref):
    w1 = w1_ref[...]
    nt = (((1,), (1,)), ((), ()))
    h = lax.dot_general(ue_ref[0], w1[:, :D], nt,
                        preferred_element_type=jnp.float32)
    h += lax.dot_general(me_ref[0], w1[:, D:], nt,
                         preferred_element_type=jnp.float32)
    h = jnp.maximum(h + c1_ref[...], 0.0)
    h = lax.dot_general(h, w2_ref[...], nt, preferred_element_type=jnp.float32)
    h = jnp.maximum(h + c2_ref[...], 0.0)
    o = jnp.sum(h * w3_ref[...], axis=1, keepdims=True) + c3_ref[...]
    out_ref[...] = 1.0 / (1.0 + jnp.exp(-o))


def kernel(users, movies, user_table, movie_table,
           W1, b1, g1, be1, W2, b2, g2, be2, W3, b3):
    u = users.astype(jnp.int32)
    m = movies.astype(jnp.int32)
    ut3 = user_table.reshape(user_table.shape[0] // 8, 8, D)
    mt3 = movie_table.reshape(movie_table.shape[0] // 8, 8, D)
    ue3, me3 = _make_sc_gather(ut3.shape[0], mt3.shape[0])(u, m, ut3, mt3)

    s = np.float32(1.0 / np.sqrt(1.0 + BN_EPS))
    w1 = W1 * (g1 * s)[:, None]
    c1 = (b1 * g1 * s + be1).reshape(1, 32)
    w2 = W2 * (g2 * s)[:, None]
    c2 = (b2 * g2 * s + be2).reshape(1, 16)
    w3 = W3.reshape(1, 16)
    c3 = b3.reshape(1, 1)

    out = pl.pallas_call(
        _mlp_body,
        grid=(NW,),
        in_specs=[
            pl.BlockSpec((1, BPW, D), lambda w: (w, 0, 0)),
            pl.BlockSpec((1, BPW, D), lambda w: (w, 0, 0)),
            pl.BlockSpec((32, 128), lambda w: (0, 0)),
            pl.BlockSpec((1, 32), lambda w: (0, 0)),
            pl.BlockSpec((16, 32), lambda w: (0, 0)),
            pl.BlockSpec((1, 16), lambda w: (0, 0)),
            pl.BlockSpec((1, 16), lambda w: (0, 0)),
            pl.BlockSpec((1, 1), lambda w: (0, 0)),
        ],
        out_specs=pl.BlockSpec((BPW, 1), lambda w: (w, 0)),
        out_shape=jax.ShapeDtypeStruct((B, 1), jnp.float32),
    )(ue3, me3, w1, c1, w2, c2, w3, c3)
    return out
